# baseline (speedup 1.0000x reference)
import functools

import jax
import jax.numpy as jnp
from jax import lax
from jax.experimental import pallas as pl
from jax.experimental.pallas import tpu as pltpu

_TOP_K = 8
_E = 64
_ALPHA = 0.1


def _router_kernel(hs_ref, w_ref, idx_ref, wt_ref, ce_ref, ss_ref, aux_ref,
                   *, blk, nsteps, blocks_per_batch, bsz, seq_len):
    i = pl.program_id(0)

    @pl.when(i == 0)
    def _init():
        ce_ref[:, :] = jnp.zeros_like(ce_ref)
        ss_ref[:, :] = jnp.zeros_like(ss_ref)

    logits = lax.dot_general(
        w_ref[:, :], hs_ref[:, :], (((1,), (1,)), ((), ())),
        preferred_element_type=jnp.float32,
        precision=lax.Precision.DEFAULT)

    m = jnp.max(logits, axis=0, keepdims=True)
    e = jnp.exp(logits - m)

    rows = lax.broadcasted_iota(jnp.int32, (_E, blk), 0)
    work = e
    vals, idxs = [], []
    for _ in range(_TOP_K):
        mx = jnp.max(work, axis=0, keepdims=True)
        pick = jnp.argmax(work, axis=0).reshape(1, blk).astype(jnp.int32)
        vals.append(mx)
        idxs.append(pick)
        work = jnp.where(rows == pick, -jnp.inf, work)

    topw = jnp.concatenate(vals, axis=0)
    topi = jnp.concatenate(idxs, axis=0)
    denom = jnp.sum(topw, axis=0, keepdims=True) + 1e-20
    wt_ref[:, :] = topw / denom
    idx_ref[:, :] = topi

    sel = (work == -jnp.inf).astype(jnp.float32)
    ones_row = jnp.ones((1, blk), jnp.float32)
    counts = lax.dot_general(
        sel, ones_row, (((1,), (1,)), ((), ())),
        preferred_element_type=jnp.float32)
    s = jnp.sum(e, axis=0, keepdims=True)
    recip_s = (1.0 / s)
    ssum = lax.dot_general(
        e, recip_s, (((1,), (1,)), ((), ())),
        preferred_element_type=jnp.float32)

    b = i // blocks_per_batch
    bcol = lax.broadcasted_iota(jnp.int32, (1, bsz), 1)
    bmask = (bcol == b).astype(jnp.float32)
    ce_ref[:, :] += counts * bmask
    ss_ref[:, :] += ssum * bmask

    @pl.when(i == nsteps - 1)
    def _fin():
        ce = ce_ref[:, :] * (_E / (seq_len * _TOP_K))
        ms = ss_ref[:, :] / seq_len
        aux_ref[:, :] = jnp.sum(ce * ms, keepdims=True).reshape(1, 1) * (_ALPHA / bsz)


def kernel(hidden_states, weight):
    bsz, seq_len, hid = hidden_states.shape
    n_tok = bsz * seq_len
    blk = 512
    nsteps = n_tok // blk
    hs = hidden_states.reshape(n_tok, hid)

    out_shapes = (
        jax.ShapeDtypeStruct((_TOP_K, n_tok), jnp.int32),
        jax.ShapeDtypeStruct((_TOP_K, n_tok), jnp.float32),
        jax.ShapeDtypeStruct((_E, bsz), jnp.float32),
        jax.ShapeDtypeStruct((_E, bsz), jnp.float32),
        jax.ShapeDtypeStruct((1, 1), jnp.float32),
    )
    in_specs = [
        pl.BlockSpec((blk, hid), lambda i: (i, 0)),
        pl.BlockSpec((_E, hid), lambda i: (0, 0)),
    ]
    out_specs = (
        pl.BlockSpec((_TOP_K, blk), lambda i: (0, i)),
        pl.BlockSpec((_TOP_K, blk), lambda i: (0, i)),
        pl.BlockSpec((_E, bsz), lambda i: (0, 0)),
        pl.BlockSpec((_E, bsz), lambda i: (0, 0)),
        pl.BlockSpec((1, 1), lambda i: (0, 0)),
    )
    idx_t, wt_t, _ce, _ss, aux = pl.pallas_call(
        functools.partial(
            _router_kernel, blk=blk, nsteps=nsteps,
            blocks_per_batch=seq_len // blk, bsz=bsz, seq_len=seq_len),
        grid=(nsteps,),
        in_specs=in_specs,
        out_specs=out_specs,
        out_shape=out_shapes,
        compiler_params=pltpu.CompilerParams(
            dimension_semantics=("arbitrary",)),
    )(hs, weight)
    return idx_t.T, wt_t.T, aux[0, 0]
